# baseline (speedup 1.0000x reference)
import functools
import math

import jax
import jax.numpy as jnp
from jax import lax
from jax.experimental import pallas as pl
from jax.experimental.pallas import tpu as pltpu
from jax.experimental.pallas import tpu_sc as plsc

_INF = float('inf')


def _r7(v):
    u = lax.bitcast_convert_type(v, jnp.int32)
    u = (u + 32767 + ((u >> 16) & 1)) & jnp.int32(-65536)
    return lax.bitcast_convert_type(u, jnp.float32)


_NW = 32


def _sc_gather(table, idx):
    B = idx.shape[0]
    D = table.shape[1]
    bpw = B // _NW
    ch = min(128, bpw)
    while bpw % ch:
        ch -= 8
    n_ch = bpw // ch
    mesh = plsc.VectorSubcoreMesh(core_axis_name="c", subcore_axis_name="s")

    @functools.partial(
        pl.kernel, mesh=mesh,
        out_type=jax.ShapeDtypeStruct((B, D), jnp.float32),
        scratch_types=[
            pltpu.VMEM((ch,), jnp.int32),
            pltpu.VMEM((ch, D), jnp.float32),
            pltpu.SemaphoreType.DMA,
        ],
    )
    def k(table_hbm, idx_hbm, out_hbm, idx_v, rows_v, sem):
        wid = lax.axis_index("s") * 2 + lax.axis_index("c")
        base = wid * bpw

        def body(g, carry):
            off = base + g * ch
            pltpu.sync_copy(idx_hbm.at[pl.ds(off, ch)], idx_v)
            pltpu.async_copy(table_hbm.at[idx_v], rows_v, sem).wait()
            pltpu.sync_copy(rows_v, out_hbm.at[pl.ds(off, ch)])
            return carry

        lax.fori_loop(0, n_ch, body, jnp.int32(0))

    return k(table, idx)


def _gather_rows(table, idx):
    B0 = idx.shape[0]
    Bp = -(-B0 // 256) * 256
    if Bp != B0:
        idx = jnp.concatenate(
            [idx, jnp.zeros((Bp - B0,), idx.dtype)])
    out = _sc_gather(table, idx)
    return out[:B0] if Bp != B0 else out


def _knn_body(qref, rref, oref, *, k, R, TQ, exclude):
    t = pl.program_id(0)
    q = qref[...]
    rt = rref[...]
    qx, qy, qz = q[:, 0:1], q[:, 1:2], q[:, 2:3]
    rx, ry, rz = rt[0:1, :], rt[1:2, :], rt[2:3, :]
    qn = (qx * qx + qy * qy) + qz * qz
    rn = (rx * rx + ry * ry) + rz * rz
    qx7, qy7, qz7 = _r7(qx), _r7(qy), _r7(qz)
    rx7, ry7, rz7 = _r7(rx), _r7(ry), _r7(rz)
    m = (qx7 * rx7 + qy7 * ry7) + qz7 * rz7
    d = (qn + rn) - 2.0 * m
    cols = lax.broadcasted_iota(jnp.int32, (TQ, R), 1)
    if exclude:
        rows = lax.broadcasted_iota(jnp.int32, (TQ, R), 0) + t * TQ
        d = jnp.where(cols == rows, _INF, d)
    outs = []
    if R >= 4096:
        CK = R // 128
        T = 6 if CK >= 128 else 8
        d3 = d.reshape(TQ, CK, 128)
        iw = lax.broadcasted_iota(jnp.int32, (TQ, CK, 128), 2)
        ck = lax.broadcasted_iota(jnp.int32, (TQ, CK), 1)
        vals, gcols = [], []
        for _ in range(T):
            v = jnp.min(d3, axis=2)
            aw = jnp.min(jnp.where(d3 == v[:, :, None], iw, jnp.int32(128)),
                         axis=2)
            vals.append(v)
            gcols.append(ck * 128 + aw)
            d3 = jnp.where(iw == aw[:, :, None], _INF, d3)
        cand = jnp.concatenate(vals, axis=1)
        gc = jnp.concatenate(gcols, axis=1)
        for _ in range(k):
            mn = jnp.min(cand, axis=1, keepdims=True)
            a = jnp.min(jnp.where(cand == mn, gc, jnp.int32(R)), axis=1,
                        keepdims=True)
            outs.append(a)
            cand = jnp.where((cand == mn) & (gc == a), _INF, cand)
    else:
        for _ in range(k):
            mn = jnp.min(d, axis=1, keepdims=True)
            a = jnp.min(jnp.where(d == mn, cols, jnp.int32(R)), axis=1,
                        keepdims=True)
            outs.append(a)
            d = jnp.where(cols == a, _INF, d)
    oref[...] = jnp.concatenate(outs, axis=1)


def _knn(query, refT, k, exclude):
    Q = query.shape[0]
    R = refT.shape[1]
    TQ = min(128, Q)
    body = functools.partial(_knn_body, k=k, R=R, TQ=TQ, exclude=exclude)
    return pl.pallas_call(
        body,
        grid=(Q // TQ,),
        in_specs=[pl.BlockSpec((TQ, 3), lambda i: (i, 0)),
                  pl.BlockSpec((3, R), lambda i: (0, 0))],
        out_specs=pl.BlockSpec((TQ, k), lambda i: (i, 0)),
        out_shape=jax.ShapeDtypeStruct((Q, k), jnp.int32),
    )(query, refT)


def _fps_body(pxr, pyr, pzr, oref, *, n_out, SR, SC, OR):
    px, py, pz = pxr[...], pyr[...], pzr[...]
    lin = (lax.broadcasted_iota(jnp.int32, (SR, SC), 0) * SC
           + lax.broadcasted_iota(jnp.int32, (SR, SC), 1))
    lane = lax.broadcasted_iota(jnp.int32, (1, SC), 1)
    oref[...] = jnp.zeros((OR, SC), jnp.int32)

    def body(i, carry):
        dists, last = carry
        msk = lin == last
        cx = jnp.sum(jnp.where(msk, px, 0.0))
        cy = jnp.sum(jnp.where(msk, py, 0.0))
        cz = jnp.sum(jnp.where(msk, pz, 0.0))
        dx, dy, dz = px - cx, py - cy, pz - cz
        d = (dx * dx + dy * dy) + dz * dz
        dists = jnp.minimum(dists, d)
        mx = jnp.max(dists)
        nxt = jnp.min(jnp.where(dists == mx, lin, jnp.int32(2147483647)))
        r = i // SC
        c = i % SC
        row = oref[pl.ds(r, 1), :]
        oref[pl.ds(r, 1), :] = jnp.where(lane == c, nxt, row)
        return dists, nxt

    dists0 = jnp.full((SR, SC), _INF, jnp.float32)
    lax.fori_loop(1, n_out, body, (dists0, jnp.int32(0)))


def _fps(p, n_out):
    Np = p.shape[0]
    SC = 128 if Np % 128 == 0 else Np
    SR = Np // SC
    OR = -(-n_out // SC)
    body = functools.partial(_fps_body, n_out=n_out, SR=SR, SC=SC, OR=OR)
    out = pl.pallas_call(
        body,
        out_shape=jax.ShapeDtypeStruct((OR, SC), jnp.int32),
    )(p[:, 0].reshape(SR, SC), p[:, 1].reshape(SR, SC),
      p[:, 2].reshape(SR, SC))
    return out.reshape(-1)[:n_out]


def _mlp_bn_body(xr, wr, br, gr, betr, oref):
    h = jnp.dot(xr[...], wr[...], preferred_element_type=jnp.float32) + br[...]
    mu = jnp.mean(h, axis=0, keepdims=True)
    xc = h - mu
    var = jnp.mean(xc * xc, axis=0, keepdims=True)
    h = xc / jnp.sqrt(var + 1e-5) * gr[...] + betr[...]
    oref[...] = jnp.maximum(h, 0.0)


def _mlp_bn(x, p):
    N = x.shape[0]
    dout = p['W'].shape[1]
    return pl.pallas_call(
        _mlp_bn_body,
        out_shape=jax.ShapeDtypeStruct((N, dout), jnp.float32),
    )(x, p['W'], p['b'].reshape(1, -1), p['gamma'].reshape(1, -1),
      p['beta'].reshape(1, -1))


def _linrelu_body(xr, wr, br, oref):
    h = jnp.dot(xr[...], wr[...], preferred_element_type=jnp.float32) + br[...]
    oref[...] = jnp.maximum(h, 0.0)


def _linrelu(x, p):
    N = x.shape[0]
    dout = p['W'].shape[1]
    return pl.pallas_call(
        _linrelu_body,
        out_shape=jax.ShapeDtypeStruct((N, dout), jnp.float32),
    )(x, p['W'], p['b'].reshape(1, -1))


def _mlp2_body(xr, w1r, b1r, w2r, b2r, oref):
    h = jnp.maximum(
        jnp.dot(xr[...], w1r[...], preferred_element_type=jnp.float32)
        + b1r[...], 0.0)
    oref[...] = (jnp.dot(h, w2r[...], preferred_element_type=jnp.float32)
                 + b2r[...])


def _mlp2_plain(x, p):
    N = x.shape[0]
    dout = p['l2']['W'].shape[1]
    return pl.pallas_call(
        _mlp2_body,
        out_shape=jax.ShapeDtypeStruct((N, dout), jnp.float32),
    )(x, p['l1']['W'], p['l1']['b'].reshape(1, -1),
      p['l2']['W'], p['l2']['b'].reshape(1, -1))


def _tpre_body(xr, wir, bir, wlr, wsr, wdr, vr, sr, dr):
    x1 = jnp.maximum(
        jnp.dot(xr[...], wir[...], preferred_element_type=jnp.float32)
        + bir[...], 0.0)
    vr[...] = jnp.dot(x1, wlr[...], preferred_element_type=jnp.float32)
    sr[...] = jnp.dot(x1, wsr[...], preferred_element_type=jnp.float32)
    dr[...] = jnp.dot(x1, wdr[...], preferred_element_type=jnp.float32)


def _tpre(x, p):
    N = x.shape[0]
    C = p['lin']['W'].shape[1]
    sh = jax.ShapeDtypeStruct((N, C), jnp.float32)
    return pl.pallas_call(
        _tpre_body,
        out_shape=[sh, sh, sh],
    )(x, p['lin_in']['W'], p['lin_in']['b'].reshape(1, -1),
      p['lin']['W'], p['lin_src']['W'], p['lin_dst']['W'])


def _edge_body(posr, pjr, xjr, ajr, adr, w1r, c1r, w2r, c2r,
               a1r, d1r, a2r, d2r, wor, cor, oref, *, K17):
    pos = posr[...]
    adst = adr[...]
    W1, b1 = w1r[...], c1r[...]
    W2, b2 = w2r[...], c2r[...]
    A1, e1 = a1r[...], d1r[...]
    A2, e2 = a2r[...], d2r[...]
    deltas = []
    als = []
    for j in range(K17):
        pd = pos - pjr[j]
        hh = jnp.maximum(
            jnp.dot(pd, W1, preferred_element_type=jnp.float32) + b1, 0.0)
        dl = jnp.maximum(
            jnp.dot(hh, W2, preferred_element_type=jnp.float32) + b2, 0.0)
        ai = (adst - ajr[j]) + dl
        h2 = jnp.maximum(
            jnp.dot(ai, A1, preferred_element_type=jnp.float32) + e1, 0.0)
        al = jnp.maximum(
            jnp.dot(h2, A2, preferred_element_type=jnp.float32) + e2, 0.0)
        deltas.append(dl)
        als.append(al)
    m = als[0]
    for j in range(1, K17):
        m = jnp.maximum(m, als[j])
    es = [jnp.exp(a - m) for a in als]
    s = es[0]
    for j in range(1, K17):
        s = s + es[j]
    o = (es[0] / s) * (xjr[0] + deltas[0])
    for j in range(1, K17):
        o = o + (es[j] / s) * (xjr[j] + deltas[j])
    oref[...] = jnp.maximum(
        jnp.dot(o, wor[...], preferred_element_type=jnp.float32) + cor[...],
        0.0)


def _edge(pos, pj, xj, aj, adst, p):
    Np, C = adst.shape
    K17 = pj.shape[0]
    TP = min(512, Np)
    body = functools.partial(_edge_body, K17=K17)
    return pl.pallas_call(
        body,
        grid=(Np // TP,),
        in_specs=[
            pl.BlockSpec((TP, 3), lambda i: (i, 0)),
            pl.BlockSpec((K17, TP, 3), lambda i: (0, i, 0)),
            pl.BlockSpec((K17, TP, C), lambda i: (0, i, 0)),
            pl.BlockSpec((K17, TP, C), lambda i: (0, i, 0)),
            pl.BlockSpec((TP, C), lambda i: (i, 0)),
            pl.BlockSpec((3, 64), lambda i: (0, 0)),
            pl.BlockSpec((1, 64), lambda i: (0, 0)),
            pl.BlockSpec((64, C), lambda i: (0, 0)),
            pl.BlockSpec((1, C), lambda i: (0, 0)),
            pl.BlockSpec((C, 64), lambda i: (0, 0)),
            pl.BlockSpec((1, 64), lambda i: (0, 0)),
            pl.BlockSpec((64, C), lambda i: (0, 0)),
            pl.BlockSpec((1, C), lambda i: (0, 0)),
            pl.BlockSpec((C, C), lambda i: (0, 0)),
            pl.BlockSpec((1, C), lambda i: (0, 0)),
        ],
        out_specs=pl.BlockSpec((TP, C), lambda i: (i, 0)),
        out_shape=jax.ShapeDtypeStruct((Np, C), jnp.float32),
    )(pos, pj, xj, aj, adst,
      p['pos_nn']['l1']['W'], p['pos_nn']['l1']['b'].reshape(1, -1),
      p['pos_nn']['l2']['W'], p['pos_nn']['l2']['b'].reshape(1, -1),
      p['attn_nn']['l1']['W'], p['attn_nn']['l1']['b'].reshape(1, -1),
      p['attn_nn']['l2']['W'], p['attn_nn']['l2']['b'].reshape(1, -1),
      p['lin_out']['W'], p['lin_out']['b'].reshape(1, -1))


def _pad_cols(a, W):
    return a if a.shape[1] == W else jnp.pad(a, ((0, 0), (0, W - a.shape[1])))


def _tblock(p, h, pos, p16, nbrs):
    v, asrc, adst = _tpre(h, p)
    Np, C = adst.shape
    K17 = nbrs.shape[1]
    Gp = -(-(2 * C + 16) // 128) * 128
    pack = _pad_cols(jnp.concatenate([v, asrc, p16], axis=1), Gp)
    rows = _gather_rows(pack, nbrs.T.reshape(-1))
    xj = rows[:, :C].reshape(K17, Np, C)
    aj = rows[:, C:2 * C].reshape(K17, Np, C)
    pj = rows[:, 2 * C:2 * C + 3].reshape(K17, Np, 3)
    return _edge(pos, pj, xj, aj, adst, p)


def _rowmax_body(gr, oref, *, KK):
    m = gr[0]
    for j in range(1, KK):
        m = jnp.maximum(m, gr[j])
    oref[...] = m


def _rowmax(g):
    KK, Ns, C = g.shape
    TP = min(512, Ns)
    body = functools.partial(_rowmax_body, KK=KK)
    return pl.pallas_call(
        body,
        grid=(Ns // TP,),
        in_specs=[pl.BlockSpec((KK, TP, C), lambda i: (0, i, 0))],
        out_specs=pl.BlockSpec((TP, C), lambda i: (i, 0)),
        out_shape=jax.ShapeDtypeStruct((Ns, C), jnp.float32),
    )(g)


def _interp_body(baser, xgr, pgr, posr, oref):
    pos = posr[...]
    num = None
    den = None
    for j in range(3):
        pd = pos - pgr[j]
        dx, dy, dz = pd[:, 0:1], pd[:, 1:2], pd[:, 2:3]
        d2 = (dx * dx + dy * dy) + dz * dz
        w = 1.0 / jnp.maximum(d2, 1e-16)
        contrib = xgr[j] * w
        num = contrib if num is None else num + contrib
        den = w if den is None else den + w
    oref[...] = baser[...] + num / den


def _interp(base, xg, pg, pos):
    Np, C = base.shape
    TP = min(512, Np)
    return pl.pallas_call(
        _interp_body,
        grid=(Np // TP,),
        in_specs=[
            pl.BlockSpec((TP, C), lambda i: (i, 0)),
            pl.BlockSpec((3, TP, C), lambda i: (0, i, 0)),
            pl.BlockSpec((3, TP, 3), lambda i: (0, i, 0)),
            pl.BlockSpec((TP, 3), lambda i: (i, 0)),
        ],
        out_specs=pl.BlockSpec((TP, C), lambda i: (i, 0)),
        out_shape=jax.ShapeDtypeStruct((Np, C), jnp.float32),
    )(base, xg, pg, pos)


def _graph_nbrs(pos):
    Np = pos.shape[0]
    idx = _knn(pos, pos.T, 16, True)
    self_col = jnp.arange(Np, dtype=idx.dtype)[:, None]
    return jnp.concatenate([idx, self_col], axis=1)


def kernel(x, pos, params):
    p128 = jnp.pad(pos, ((0, 0), (0, 125)))
    h = _mlp_bn(x, params['mlp_input'])
    nbrs = _graph_nbrs(pos)
    h = _tblock(params['t_in'], h, pos, p128[:, :16], nbrs)
    out_x, out_pos, out_p128, out_nbrs = [h], [pos], [p128], [nbrs]
    p = pos
    for i in range(4):
        n_out = int(math.ceil(p.shape[0] * 0.25))
        sel = _fps(p, n_out)
        p128 = _gather_rows(p128, sel)
        p_sub = p128[:, :3]
        idx = _knn(p_sub, p.T, 16, False)
        hm = _mlp_bn(h, params['td'][i])
        C = hm.shape[1]
        Cp = max(C, 128)
        g = _gather_rows(_pad_cols(hm, Cp), idx.T.reshape(-1))
        h = _rowmax(g.reshape(16, n_out, Cp))[:, :C]
        p = p_sub
        nbrs = _graph_nbrs(p)
        h = _tblock(params['tf_down'][i], h, p, p128[:, :16], nbrs)
        out_x.append(h)
        out_pos.append(p)
        out_p128.append(p128)
        out_nbrs.append(nbrs)
    h = _linrelu(h, params['mlp_summit'])
    h = _tblock(params['t_summit'], h, p, p128[:, :16], out_nbrs[-1])
    for i in range(4):
        x_skip = out_x[-i - 2]
        pos_skip = out_pos[-i - 2]
        pos_sub = out_pos[-i - 1]
        tu = params['tu'][3 - i]
        h_sub = _mlp_bn(h, tu['mlp_sub'])
        idx3 = _knn(pos_skip, pos_sub.T, 3, False)
        Np = pos_skip.shape[0]
        C = h_sub.shape[1]
        Gp = -(-(C + 16) // 128) * 128
        pack = _pad_cols(
            jnp.concatenate([h_sub, out_p128[-i - 1][:, :16]], axis=1), Gp)
        rows = _gather_rows(pack, idx3.T.reshape(-1))
        xg = rows[:, :C].reshape(3, Np, C)
        pg = rows[:, C:C + 3].reshape(3, Np, 3)
        base = _mlp_bn(x_skip, tu['mlp'])
        h = _interp(base, xg, pg, pos_skip)
        h = _tblock(params['tf_up'][3 - i], h, pos_skip,
                    out_p128[-i - 2][:, :16], out_nbrs[-i - 2])
    return _mlp2_plain(h, params['mlp_out'])


# speedup vs baseline: 1.2656x; 1.2656x over previous
import functools
import math

import jax
import jax.numpy as jnp
from jax import lax
from jax.experimental import pallas as pl
from jax.experimental.pallas import tpu as pltpu
from jax.experimental.pallas import tpu_sc as plsc

_INF = float('inf')


def _r7(v):
    u = lax.bitcast_convert_type(v, jnp.int32)
    u = (u + 32767 + ((u >> 16) & 1)) & jnp.int32(-65536)
    return lax.bitcast_convert_type(u, jnp.float32)


_NW = 32


def _sc_gather(table, idx):
    B = idx.shape[0]
    D = table.shape[1]
    bpw = B // _NW
    ch = min(128, bpw)
    while bpw % ch:
        ch -= 8
    n_ch = bpw // ch
    mesh = plsc.VectorSubcoreMesh(core_axis_name="c", subcore_axis_name="s")

    @functools.partial(
        pl.kernel, mesh=mesh,
        out_type=jax.ShapeDtypeStruct((B, D), jnp.float32),
        scratch_types=[
            pltpu.VMEM((ch,), jnp.int32),
            pltpu.VMEM((ch, D), jnp.float32),
            pltpu.SemaphoreType.DMA,
        ],
    )
    def k(table_hbm, idx_hbm, out_hbm, idx_v, rows_v, sem):
        wid = lax.axis_index("s") * 2 + lax.axis_index("c")
        base = wid * bpw

        def body(g, carry):
            off = base + g * ch
            pltpu.sync_copy(idx_hbm.at[pl.ds(off, ch)], idx_v)
            pltpu.async_copy(table_hbm.at[idx_v], rows_v, sem).wait()
            pltpu.sync_copy(rows_v, out_hbm.at[pl.ds(off, ch)])
            return carry

        lax.fori_loop(0, n_ch, body, jnp.int32(0))

    return k(table, idx)


def _gather_rows(table, idx):
    B0 = idx.shape[0]
    Bp = -(-B0 // 256) * 256
    if Bp != B0:
        idx = jnp.concatenate(
            [idx, jnp.zeros((Bp - B0,), idx.dtype)])
    out = _sc_gather(table, idx)
    return out[:B0] if Bp != B0 else out


def _knn_body(qref, rref, oref, *, k, R, TQ, exclude):
    t = pl.program_id(0)
    q = qref[...]
    rt = rref[...]
    qx, qy, qz = q[:, 0:1], q[:, 1:2], q[:, 2:3]
    rx, ry, rz = rt[0:1, :], rt[1:2, :], rt[2:3, :]
    qn = (qx * qx + qy * qy) + qz * qz
    rn = (rx * rx + ry * ry) + rz * rz
    qx7, qy7, qz7 = _r7(qx), _r7(qy), _r7(qz)
    rx7, ry7, rz7 = _r7(rx), _r7(ry), _r7(rz)
    m = (qx7 * rx7 + qy7 * ry7) + qz7 * rz7
    d = (qn + rn) - 2.0 * m
    cols = lax.broadcasted_iota(jnp.int32, (TQ, R), 1)
    if exclude:
        rows = lax.broadcasted_iota(jnp.int32, (TQ, R), 0) + t * TQ
        d = jnp.where(cols == rows, _INF, d)
    outs = []
    for _ in range(k):
        a = jnp.argmin(d, axis=1).astype(jnp.int32)[:, None]
        outs.append(a)
        d = jnp.where(cols == a, _INF, d)
    oref[...] = jnp.concatenate(outs, axis=1)


def _knn(query, refT, k, exclude):
    Q = query.shape[0]
    R = refT.shape[1]
    TQ = min(128, Q)
    body = functools.partial(_knn_body, k=k, R=R, TQ=TQ, exclude=exclude)
    return pl.pallas_call(
        body,
        grid=(Q // TQ,),
        in_specs=[pl.BlockSpec((TQ, 3), lambda i: (i, 0)),
                  pl.BlockSpec((3, R), lambda i: (0, 0))],
        out_specs=pl.BlockSpec((TQ, k), lambda i: (i, 0)),
        out_shape=jax.ShapeDtypeStruct((Q, k), jnp.int32),
    )(query, refT)


def _fps_body(pxr, pyr, pzr, oref, *, n_out, SR, SC, OR):
    px, py, pz = pxr[...], pyr[...], pzr[...]
    lin = (lax.broadcasted_iota(jnp.int32, (SR, SC), 0) * SC
           + lax.broadcasted_iota(jnp.int32, (SR, SC), 1))
    lane = lax.broadcasted_iota(jnp.int32, (1, SC), 1)
    oref[...] = jnp.zeros((OR, SC), jnp.int32)

    def body(i, carry):
        dists, last = carry
        msk = lin == last
        cx = jnp.sum(jnp.where(msk, px, 0.0))
        cy = jnp.sum(jnp.where(msk, py, 0.0))
        cz = jnp.sum(jnp.where(msk, pz, 0.0))
        dx, dy, dz = px - cx, py - cy, pz - cz
        d = (dx * dx + dy * dy) + dz * dz
        dists = jnp.minimum(dists, d)
        mx = jnp.max(dists)
        nxt = jnp.min(jnp.where(dists == mx, lin, jnp.int32(2147483647)))
        r = i // SC
        c = i % SC
        row = oref[pl.ds(r, 1), :]
        oref[pl.ds(r, 1), :] = jnp.where(lane == c, nxt, row)
        return dists, nxt

    dists0 = jnp.full((SR, SC), _INF, jnp.float32)
    lax.fori_loop(1, n_out, body, (dists0, jnp.int32(0)))


def _fps(p, n_out):
    Np = p.shape[0]
    SC = 128 if Np % 128 == 0 else Np
    SR = Np // SC
    OR = -(-n_out // SC)
    body = functools.partial(_fps_body, n_out=n_out, SR=SR, SC=SC, OR=OR)
    out = pl.pallas_call(
        body,
        out_shape=jax.ShapeDtypeStruct((OR, SC), jnp.int32),
    )(p[:, 0].reshape(SR, SC), p[:, 1].reshape(SR, SC),
      p[:, 2].reshape(SR, SC))
    return out.reshape(-1)[:n_out]


def _mlp_bn_body(xr, wr, br, gr, betr, oref):
    h = jnp.dot(xr[...], wr[...], preferred_element_type=jnp.float32) + br[...]
    mu = jnp.mean(h, axis=0, keepdims=True)
    xc = h - mu
    var = jnp.mean(xc * xc, axis=0, keepdims=True)
    h = xc / jnp.sqrt(var + 1e-5) * gr[...] + betr[...]
    oref[...] = jnp.maximum(h, 0.0)


def _mlp_bn(x, p):
    N = x.shape[0]
    dout = p['W'].shape[1]
    return pl.pallas_call(
        _mlp_bn_body,
        out_shape=jax.ShapeDtypeStruct((N, dout), jnp.float32),
    )(x, p['W'], p['b'].reshape(1, -1), p['gamma'].reshape(1, -1),
      p['beta'].reshape(1, -1))


def _linrelu_body(xr, wr, br, oref):
    h = jnp.dot(xr[...], wr[...], preferred_element_type=jnp.float32) + br[...]
    oref[...] = jnp.maximum(h, 0.0)


def _linrelu(x, p):
    N = x.shape[0]
    dout = p['W'].shape[1]
    return pl.pallas_call(
        _linrelu_body,
        out_shape=jax.ShapeDtypeStruct((N, dout), jnp.float32),
    )(x, p['W'], p['b'].reshape(1, -1))


def _mlp2_body(xr, w1r, b1r, w2r, b2r, oref):
    h = jnp.maximum(
        jnp.dot(xr[...], w1r[...], preferred_element_type=jnp.float32)
        + b1r[...], 0.0)
    oref[...] = (jnp.dot(h, w2r[...], preferred_element_type=jnp.float32)
                 + b2r[...])


def _mlp2_plain(x, p):
    N = x.shape[0]
    dout = p['l2']['W'].shape[1]
    return pl.pallas_call(
        _mlp2_body,
        out_shape=jax.ShapeDtypeStruct((N, dout), jnp.float32),
    )(x, p['l1']['W'], p['l1']['b'].reshape(1, -1),
      p['l2']['W'], p['l2']['b'].reshape(1, -1))


def _tpre_body(xr, wir, bir, wlr, wsr, wdr, vr, sr, dr):
    x1 = jnp.maximum(
        jnp.dot(xr[...], wir[...], preferred_element_type=jnp.float32)
        + bir[...], 0.0)
    vr[...] = jnp.dot(x1, wlr[...], preferred_element_type=jnp.float32)
    sr[...] = jnp.dot(x1, wsr[...], preferred_element_type=jnp.float32)
    dr[...] = jnp.dot(x1, wdr[...], preferred_element_type=jnp.float32)


def _tpre(x, p):
    N = x.shape[0]
    C = p['lin']['W'].shape[1]
    sh = jax.ShapeDtypeStruct((N, C), jnp.float32)
    return pl.pallas_call(
        _tpre_body,
        out_shape=[sh, sh, sh],
    )(x, p['lin_in']['W'], p['lin_in']['b'].reshape(1, -1),
      p['lin']['W'], p['lin_src']['W'], p['lin_dst']['W'])


def _edge_body(posr, pjr, xjr, ajr, adr, w1r, c1r, w2r, c2r,
               a1r, d1r, a2r, d2r, wor, cor, oref, *, K17):
    pos = posr[...]
    adst = adr[...]
    W1, b1 = w1r[...], c1r[...]
    W2, b2 = w2r[...], c2r[...]
    A1, e1 = a1r[...], d1r[...]
    A2, e2 = a2r[...], d2r[...]
    deltas = []
    als = []
    for j in range(K17):
        pd = pos - pjr[j]
        hh = jnp.maximum(
            jnp.dot(pd, W1, preferred_element_type=jnp.float32) + b1, 0.0)
        dl = jnp.maximum(
            jnp.dot(hh, W2, preferred_element_type=jnp.float32) + b2, 0.0)
        ai = (adst - ajr[j]) + dl
        h2 = jnp.maximum(
            jnp.dot(ai, A1, preferred_element_type=jnp.float32) + e1, 0.0)
        al = jnp.maximum(
            jnp.dot(h2, A2, preferred_element_type=jnp.float32) + e2, 0.0)
        deltas.append(dl)
        als.append(al)
    m = als[0]
    for j in range(1, K17):
        m = jnp.maximum(m, als[j])
    es = [jnp.exp(a - m) for a in als]
    s = es[0]
    for j in range(1, K17):
        s = s + es[j]
    o = (es[0] / s) * (xjr[0] + deltas[0])
    for j in range(1, K17):
        o = o + (es[j] / s) * (xjr[j] + deltas[j])
    oref[...] = jnp.maximum(
        jnp.dot(o, wor[...], preferred_element_type=jnp.float32) + cor[...],
        0.0)


def _edge(pos, pj, xj, aj, adst, p):
    Np, C = adst.shape
    K17 = pj.shape[0]
    TP = min(512, Np)
    body = functools.partial(_edge_body, K17=K17)
    return pl.pallas_call(
        body,
        grid=(Np // TP,),
        in_specs=[
            pl.BlockSpec((TP, 3), lambda i: (i, 0)),
            pl.BlockSpec((K17, TP, 3), lambda i: (0, i, 0)),
            pl.BlockSpec((K17, TP, C), lambda i: (0, i, 0)),
            pl.BlockSpec((K17, TP, C), lambda i: (0, i, 0)),
            pl.BlockSpec((TP, C), lambda i: (i, 0)),
            pl.BlockSpec((3, 64), lambda i: (0, 0)),
            pl.BlockSpec((1, 64), lambda i: (0, 0)),
            pl.BlockSpec((64, C), lambda i: (0, 0)),
            pl.BlockSpec((1, C), lambda i: (0, 0)),
            pl.BlockSpec((C, 64), lambda i: (0, 0)),
            pl.BlockSpec((1, 64), lambda i: (0, 0)),
            pl.BlockSpec((64, C), lambda i: (0, 0)),
            pl.BlockSpec((1, C), lambda i: (0, 0)),
            pl.BlockSpec((C, C), lambda i: (0, 0)),
            pl.BlockSpec((1, C), lambda i: (0, 0)),
        ],
        out_specs=pl.BlockSpec((TP, C), lambda i: (i, 0)),
        out_shape=jax.ShapeDtypeStruct((Np, C), jnp.float32),
    )(pos, pj, xj, aj, adst,
      p['pos_nn']['l1']['W'], p['pos_nn']['l1']['b'].reshape(1, -1),
      p['pos_nn']['l2']['W'], p['pos_nn']['l2']['b'].reshape(1, -1),
      p['attn_nn']['l1']['W'], p['attn_nn']['l1']['b'].reshape(1, -1),
      p['attn_nn']['l2']['W'], p['attn_nn']['l2']['b'].reshape(1, -1),
      p['lin_out']['W'], p['lin_out']['b'].reshape(1, -1))


def _pad_cols(a, W):
    return a if a.shape[1] == W else jnp.pad(a, ((0, 0), (0, W - a.shape[1])))


def _tblock(p, h, pos, p16, nbrs):
    v, asrc, adst = _tpre(h, p)
    Np, C = adst.shape
    K17 = nbrs.shape[1]
    Gp = -(-(2 * C + 16) // 128) * 128
    pack = _pad_cols(jnp.concatenate([v, asrc, p16], axis=1), Gp)
    rows = _gather_rows(pack, nbrs.T.reshape(-1))
    xj = rows[:, :C].reshape(K17, Np, C)
    aj = rows[:, C:2 * C].reshape(K17, Np, C)
    pj = rows[:, 2 * C:2 * C + 3].reshape(K17, Np, 3)
    return _edge(pos, pj, xj, aj, adst, p)


def _rowmax_body(gr, oref, *, KK):
    m = gr[0]
    for j in range(1, KK):
        m = jnp.maximum(m, gr[j])
    oref[...] = m


def _rowmax(g):
    KK, Ns, C = g.shape
    TP = min(512, Ns)
    body = functools.partial(_rowmax_body, KK=KK)
    return pl.pallas_call(
        body,
        grid=(Ns // TP,),
        in_specs=[pl.BlockSpec((KK, TP, C), lambda i: (0, i, 0))],
        out_specs=pl.BlockSpec((TP, C), lambda i: (i, 0)),
        out_shape=jax.ShapeDtypeStruct((Ns, C), jnp.float32),
    )(g)


def _interp_body(baser, xgr, pgr, posr, oref):
    pos = posr[...]
    num = None
    den = None
    for j in range(3):
        pd = pos - pgr[j]
        dx, dy, dz = pd[:, 0:1], pd[:, 1:2], pd[:, 2:3]
        d2 = (dx * dx + dy * dy) + dz * dz
        w = 1.0 / jnp.maximum(d2, 1e-16)
        contrib = xgr[j] * w
        num = contrib if num is None else num + contrib
        den = w if den is None else den + w
    oref[...] = baser[...] + num / den


def _interp(base, xg, pg, pos):
    Np, C = base.shape
    TP = min(512, Np)
    return pl.pallas_call(
        _interp_body,
        grid=(Np // TP,),
        in_specs=[
            pl.BlockSpec((TP, C), lambda i: (i, 0)),
            pl.BlockSpec((3, TP, C), lambda i: (0, i, 0)),
            pl.BlockSpec((3, TP, 3), lambda i: (0, i, 0)),
            pl.BlockSpec((TP, 3), lambda i: (i, 0)),
        ],
        out_specs=pl.BlockSpec((TP, C), lambda i: (i, 0)),
        out_shape=jax.ShapeDtypeStruct((Np, C), jnp.float32),
    )(base, xg, pg, pos)


def _graph_nbrs(pos):
    Np = pos.shape[0]
    idx = _knn(pos, pos.T, 16, True)
    self_col = jnp.arange(Np, dtype=idx.dtype)[:, None]
    return jnp.concatenate([idx, self_col], axis=1)


def kernel(x, pos, params):
    p128 = jnp.pad(pos, ((0, 0), (0, 125)))
    h = _mlp_bn(x, params['mlp_input'])
    nbrs = _graph_nbrs(pos)
    h = _tblock(params['t_in'], h, pos, p128[:, :16], nbrs)
    out_x, out_pos, out_p128, out_nbrs = [h], [pos], [p128], [nbrs]
    p = pos
    for i in range(4):
        n_out = int(math.ceil(p.shape[0] * 0.25))
        sel = _fps(p, n_out)
        p128 = _gather_rows(p128, sel)
        p_sub = p128[:, :3]
        idx = _knn(p_sub, p.T, 16, False)
        hm = _mlp_bn(h, params['td'][i])
        C = hm.shape[1]
        Cp = max(C, 128)
        g = _gather_rows(_pad_cols(hm, Cp), idx.T.reshape(-1))
        h = _rowmax(g.reshape(16, n_out, Cp))[:, :C]
        p = p_sub
        nbrs = _graph_nbrs(p)
        h = _tblock(params['tf_down'][i], h, p, p128[:, :16], nbrs)
        out_x.append(h)
        out_pos.append(p)
        out_p128.append(p128)
        out_nbrs.append(nbrs)
    h = _linrelu(h, params['mlp_summit'])
    h = _tblock(params['t_summit'], h, p, p128[:, :16], out_nbrs[-1])
    for i in range(4):
        x_skip = out_x[-i - 2]
        pos_skip = out_pos[-i - 2]
        pos_sub = out_pos[-i - 1]
        tu = params['tu'][3 - i]
        h_sub = _mlp_bn(h, tu['mlp_sub'])
        idx3 = _knn(pos_skip, pos_sub.T, 3, False)
        Np = pos_skip.shape[0]
        C = h_sub.shape[1]
        Gp = -(-(C + 16) // 128) * 128
        pack = _pad_cols(
            jnp.concatenate([h_sub, out_p128[-i - 1][:, :16]], axis=1), Gp)
        rows = _gather_rows(pack, idx3.T.reshape(-1))
        xg = rows[:, :C].reshape(3, Np, C)
        pg = rows[:, C:C + 3].reshape(3, Np, 3)
        base = _mlp_bn(x_skip, tu['mlp'])
        h = _interp(base, xg, pg, pos_skip)
        h = _tblock(params['tf_up'][3 - i], h, pos_skip,
                    out_p128[-i - 2][:, :16], out_nbrs[-i - 2])
    return _mlp2_plain(h, params['mlp_out'])


# speedup vs baseline: 1.2843x; 1.0148x over previous
import functools
import math

import jax
import jax.numpy as jnp
from jax import lax
from jax.experimental import pallas as pl
from jax.experimental.pallas import tpu as pltpu
from jax.experimental.pallas import tpu_sc as plsc

_INF = float('inf')


def _r7(v):
    u = lax.bitcast_convert_type(v, jnp.int32)
    u = (u + 32767 + ((u >> 16) & 1)) & jnp.int32(-65536)
    return lax.bitcast_convert_type(u, jnp.float32)


_NW = 32


def _sc_gather(table, idx):
    B = idx.shape[0]
    D = table.shape[1]
    bpw = B // _NW
    ch = min(128, bpw)
    while bpw % ch:
        ch -= 8
    n_ch = bpw // ch
    mesh = plsc.VectorSubcoreMesh(core_axis_name="c", subcore_axis_name="s")

    @functools.partial(
        pl.kernel, mesh=mesh,
        out_type=jax.ShapeDtypeStruct((B, D), jnp.float32),
        scratch_types=[
            pltpu.VMEM((ch,), jnp.int32),
            pltpu.VMEM((ch, D), jnp.float32),
            pltpu.SemaphoreType.DMA,
        ],
    )
    def k(table_hbm, idx_hbm, out_hbm, idx_v, rows_v, sem):
        wid = lax.axis_index("s") * 2 + lax.axis_index("c")
        base = wid * bpw

        def body(g, carry):
            off = base + g * ch
            pltpu.sync_copy(idx_hbm.at[pl.ds(off, ch)], idx_v)
            pltpu.async_copy(table_hbm.at[idx_v], rows_v, sem).wait()
            pltpu.sync_copy(rows_v, out_hbm.at[pl.ds(off, ch)])
            return carry

        lax.fori_loop(0, n_ch, body, jnp.int32(0))

    return k(table, idx)


def _gather_rows(table, idx):
    B0 = idx.shape[0]
    Bp = -(-B0 // 256) * 256
    if Bp != B0:
        idx = jnp.concatenate(
            [idx, jnp.zeros((Bp - B0,), idx.dtype)])
    out = _sc_gather(table, idx)
    return out[:B0] if Bp != B0 else out


def _knn_body(qref, rref, oref, *, k, R, TQ, exclude):
    t = pl.program_id(0)
    q = qref[...]
    rt = rref[...]
    qx, qy, qz = q[:, 0:1], q[:, 1:2], q[:, 2:3]
    rx, ry, rz = rt[0:1, :], rt[1:2, :], rt[2:3, :]
    qn = (qx * qx + qy * qy) + qz * qz
    rn = (rx * rx + ry * ry) + rz * rz
    qx7, qy7, qz7 = _r7(qx), _r7(qy), _r7(qz)
    rx7, ry7, rz7 = _r7(rx), _r7(ry), _r7(rz)
    m = (qx7 * rx7 + qy7 * ry7) + qz7 * rz7
    d = (qn + rn) - 2.0 * m
    cols = lax.broadcasted_iota(jnp.int32, (TQ, R), 1)
    if exclude:
        rows = lax.broadcasted_iota(jnp.int32, (TQ, R), 0) + t * TQ
        d = jnp.where(cols == rows, _INF, d)
    outs = []
    for _ in range(k):
        a = jnp.argmin(d, axis=1).astype(jnp.int32)[:, None]
        outs.append(a)
        d = jnp.where(cols == a, _INF, d)
    oref[...] = jnp.concatenate(outs, axis=1)


def _knn(query, refT, k, exclude):
    Q = query.shape[0]
    R = refT.shape[1]
    TQ = min(128, Q)
    body = functools.partial(_knn_body, k=k, R=R, TQ=TQ, exclude=exclude)
    return pl.pallas_call(
        body,
        grid=(Q // TQ,),
        in_specs=[pl.BlockSpec((TQ, 3), lambda i: (i, 0)),
                  pl.BlockSpec((3, R), lambda i: (0, 0))],
        out_specs=pl.BlockSpec((TQ, k), lambda i: (i, 0)),
        out_shape=jax.ShapeDtypeStruct((Q, k), jnp.int32),
    )(query, refT)


def _fps_body(pxr, pyr, pzr, oref, *, n_out, SR, SC, OR):
    px, py, pz = pxr[...], pyr[...], pzr[...]
    lin = (lax.broadcasted_iota(jnp.int32, (SR, SC), 0) * SC
           + lax.broadcasted_iota(jnp.int32, (SR, SC), 1))
    lane = lax.broadcasted_iota(jnp.int32, (1, SC), 1)
    oref[...] = jnp.zeros((OR, SC), jnp.int32)

    def body(i, carry):
        dists, last = carry
        msk = lin == last
        cx = jnp.sum(jnp.where(msk, px, 0.0))
        cy = jnp.sum(jnp.where(msk, py, 0.0))
        cz = jnp.sum(jnp.where(msk, pz, 0.0))
        dx, dy, dz = px - cx, py - cy, pz - cz
        d = (dx * dx + dy * dy) + dz * dz
        dists = jnp.minimum(dists, d)
        mx = jnp.max(dists)
        nxt = jnp.min(jnp.where(dists == mx, lin, jnp.int32(2147483647)))
        r = i // SC
        c = i % SC
        row = oref[pl.ds(r, 1), :]
        oref[pl.ds(r, 1), :] = jnp.where(lane == c, nxt, row)
        return dists, nxt

    dists0 = jnp.full((SR, SC), _INF, jnp.float32)
    lax.fori_loop(1, n_out, body, (dists0, jnp.int32(0)))


def _fps(p, n_out):
    Np = p.shape[0]
    SC = 128 if Np % 128 == 0 else Np
    SR = Np // SC
    OR = -(-n_out // SC)
    body = functools.partial(_fps_body, n_out=n_out, SR=SR, SC=SC, OR=OR)
    out = pl.pallas_call(
        body,
        out_shape=jax.ShapeDtypeStruct((OR, SC), jnp.int32),
    )(p[:, 0].reshape(SR, SC), p[:, 1].reshape(SR, SC),
      p[:, 2].reshape(SR, SC))
    return out.reshape(-1)[:n_out]


def _mlp_bn_body(xr, wr, br, gr, betr, oref):
    h = jnp.dot(xr[...], wr[...], preferred_element_type=jnp.float32) + br[...]
    mu = jnp.mean(h, axis=0, keepdims=True)
    xc = h - mu
    var = jnp.mean(xc * xc, axis=0, keepdims=True)
    h = xc / jnp.sqrt(var + 1e-5) * gr[...] + betr[...]
    oref[...] = jnp.maximum(h, 0.0)


def _mlp_bn(x, p):
    N = x.shape[0]
    dout = p['W'].shape[1]
    return pl.pallas_call(
        _mlp_bn_body,
        out_shape=jax.ShapeDtypeStruct((N, dout), jnp.float32),
    )(x, p['W'], p['b'].reshape(1, -1), p['gamma'].reshape(1, -1),
      p['beta'].reshape(1, -1))


def _linrelu_body(xr, wr, br, oref):
    h = jnp.dot(xr[...], wr[...], preferred_element_type=jnp.float32) + br[...]
    oref[...] = jnp.maximum(h, 0.0)


def _linrelu(x, p):
    N = x.shape[0]
    dout = p['W'].shape[1]
    return pl.pallas_call(
        _linrelu_body,
        out_shape=jax.ShapeDtypeStruct((N, dout), jnp.float32),
    )(x, p['W'], p['b'].reshape(1, -1))


def _mlp2_body(xr, w1r, b1r, w2r, b2r, oref):
    h = jnp.maximum(
        jnp.dot(xr[...], w1r[...], preferred_element_type=jnp.float32)
        + b1r[...], 0.0)
    oref[...] = (jnp.dot(h, w2r[...], preferred_element_type=jnp.float32)
                 + b2r[...])


def _mlp2_plain(x, p):
    N = x.shape[0]
    dout = p['l2']['W'].shape[1]
    return pl.pallas_call(
        _mlp2_body,
        out_shape=jax.ShapeDtypeStruct((N, dout), jnp.float32),
    )(x, p['l1']['W'], p['l1']['b'].reshape(1, -1),
      p['l2']['W'], p['l2']['b'].reshape(1, -1))


def _tpre_body(xr, wir, bir, wlr, wsr, wdr, vr, sr, dr):
    x1 = jnp.maximum(
        jnp.dot(xr[...], wir[...], preferred_element_type=jnp.float32)
        + bir[...], 0.0)
    vr[...] = jnp.dot(x1, wlr[...], preferred_element_type=jnp.float32)
    sr[...] = jnp.dot(x1, wsr[...], preferred_element_type=jnp.float32)
    dr[...] = jnp.dot(x1, wdr[...], preferred_element_type=jnp.float32)


def _tpre(x, p):
    N = x.shape[0]
    C = p['lin']['W'].shape[1]
    sh = jax.ShapeDtypeStruct((N, C), jnp.float32)
    return pl.pallas_call(
        _tpre_body,
        out_shape=[sh, sh, sh],
    )(x, p['lin_in']['W'], p['lin_in']['b'].reshape(1, -1),
      p['lin']['W'], p['lin_src']['W'], p['lin_dst']['W'])


def _edge_body(posr, pjr, xjr, ajr, adr, w1r, c1r, w2r, c2r,
               a1r, d1r, a2r, d2r, wor, cor, oref, *, K17):
    pos = posr[...]
    adst = adr[...]
    W1, b1 = w1r[...], c1r[...]
    W2, b2 = w2r[...], c2r[...]
    A1, e1 = a1r[...], d1r[...]
    A2, e2 = a2r[...], d2r[...]
    TP = pos.shape[0]
    C = adst.shape[1]
    E = K17 * TP
    pj = pjr[...].reshape(E, 3)
    xj = xjr[...].reshape(E, C)
    aj = ajr[...].reshape(E, C)
    posb = jnp.concatenate([pos] * K17, axis=0)
    adb = jnp.concatenate([adst] * K17, axis=0)
    pd = posb - pj
    h1 = jnp.maximum(
        jnp.dot(pd, W1, preferred_element_type=jnp.float32) + b1, 0.0)
    dl = jnp.maximum(
        jnp.dot(h1, W2, preferred_element_type=jnp.float32) + b2, 0.0)
    ai = (adb - aj) + dl
    h2 = jnp.maximum(
        jnp.dot(ai, A1, preferred_element_type=jnp.float32) + e1, 0.0)
    al = jnp.maximum(
        jnp.dot(h2, A2, preferred_element_type=jnp.float32) + e2, 0.0)
    m = al[0:TP]
    for j in range(1, K17):
        m = jnp.maximum(m, al[j * TP:(j + 1) * TP])
    es = [jnp.exp(al[j * TP:(j + 1) * TP] - m) for j in range(K17)]
    s = es[0]
    for j in range(1, K17):
        s = s + es[j]
    con = xj + dl
    o = (es[0] / s) * con[0:TP]
    for j in range(1, K17):
        o = o + (es[j] / s) * con[j * TP:(j + 1) * TP]
    oref[...] = jnp.maximum(
        jnp.dot(o, wor[...], preferred_element_type=jnp.float32) + cor[...],
        0.0)


def _edge(pos, pj, xj, aj, adst, p):
    Np, C = adst.shape
    K17 = pj.shape[0]
    TP = min(512, Np)
    body = functools.partial(_edge_body, K17=K17)
    return pl.pallas_call(
        body,
        grid=(Np // TP,),
        in_specs=[
            pl.BlockSpec((TP, 3), lambda i: (i, 0)),
            pl.BlockSpec((K17, TP, 3), lambda i: (0, i, 0)),
            pl.BlockSpec((K17, TP, C), lambda i: (0, i, 0)),
            pl.BlockSpec((K17, TP, C), lambda i: (0, i, 0)),
            pl.BlockSpec((TP, C), lambda i: (i, 0)),
            pl.BlockSpec((3, 64), lambda i: (0, 0)),
            pl.BlockSpec((1, 64), lambda i: (0, 0)),
            pl.BlockSpec((64, C), lambda i: (0, 0)),
            pl.BlockSpec((1, C), lambda i: (0, 0)),
            pl.BlockSpec((C, 64), lambda i: (0, 0)),
            pl.BlockSpec((1, 64), lambda i: (0, 0)),
            pl.BlockSpec((64, C), lambda i: (0, 0)),
            pl.BlockSpec((1, C), lambda i: (0, 0)),
            pl.BlockSpec((C, C), lambda i: (0, 0)),
            pl.BlockSpec((1, C), lambda i: (0, 0)),
        ],
        out_specs=pl.BlockSpec((TP, C), lambda i: (i, 0)),
        out_shape=jax.ShapeDtypeStruct((Np, C), jnp.float32),
    )(pos, pj, xj, aj, adst,
      p['pos_nn']['l1']['W'], p['pos_nn']['l1']['b'].reshape(1, -1),
      p['pos_nn']['l2']['W'], p['pos_nn']['l2']['b'].reshape(1, -1),
      p['attn_nn']['l1']['W'], p['attn_nn']['l1']['b'].reshape(1, -1),
      p['attn_nn']['l2']['W'], p['attn_nn']['l2']['b'].reshape(1, -1),
      p['lin_out']['W'], p['lin_out']['b'].reshape(1, -1))


def _pad_cols(a, W):
    return a if a.shape[1] == W else jnp.pad(a, ((0, 0), (0, W - a.shape[1])))


def _tblock(p, h, pos, p16, nbrs):
    v, asrc, adst = _tpre(h, p)
    Np, C = adst.shape
    K17 = nbrs.shape[1]
    Gp = -(-(2 * C + 16) // 128) * 128
    pack = _pad_cols(jnp.concatenate([v, asrc, p16], axis=1), Gp)
    rows = _gather_rows(pack, nbrs.T.reshape(-1))
    xj = rows[:, :C].reshape(K17, Np, C)
    aj = rows[:, C:2 * C].reshape(K17, Np, C)
    pj = rows[:, 2 * C:2 * C + 3].reshape(K17, Np, 3)
    return _edge(pos, pj, xj, aj, adst, p)


def _rowmax_body(gr, oref, *, KK):
    m = gr[0]
    for j in range(1, KK):
        m = jnp.maximum(m, gr[j])
    oref[...] = m


def _rowmax(g):
    KK, Ns, C = g.shape
    TP = min(512, Ns)
    body = functools.partial(_rowmax_body, KK=KK)
    return pl.pallas_call(
        body,
        grid=(Ns // TP,),
        in_specs=[pl.BlockSpec((KK, TP, C), lambda i: (0, i, 0))],
        out_specs=pl.BlockSpec((TP, C), lambda i: (i, 0)),
        out_shape=jax.ShapeDtypeStruct((Ns, C), jnp.float32),
    )(g)


def _interp_body(baser, xgr, pgr, posr, oref):
    pos = posr[...]
    num = None
    den = None
    for j in range(3):
        pd = pos - pgr[j]
        dx, dy, dz = pd[:, 0:1], pd[:, 1:2], pd[:, 2:3]
        d2 = (dx * dx + dy * dy) + dz * dz
        w = 1.0 / jnp.maximum(d2, 1e-16)
        contrib = xgr[j] * w
        num = contrib if num is None else num + contrib
        den = w if den is None else den + w
    oref[...] = baser[...] + num / den


def _interp(base, xg, pg, pos):
    Np, C = base.shape
    TP = min(512, Np)
    return pl.pallas_call(
        _interp_body,
        grid=(Np // TP,),
        in_specs=[
            pl.BlockSpec((TP, C), lambda i: (i, 0)),
            pl.BlockSpec((3, TP, C), lambda i: (0, i, 0)),
            pl.BlockSpec((3, TP, 3), lambda i: (0, i, 0)),
            pl.BlockSpec((TP, 3), lambda i: (i, 0)),
        ],
        out_specs=pl.BlockSpec((TP, C), lambda i: (i, 0)),
        out_shape=jax.ShapeDtypeStruct((Np, C), jnp.float32),
    )(base, xg, pg, pos)


def _graph_nbrs(pos):
    Np = pos.shape[0]
    idx = _knn(pos, pos.T, 16, True)
    self_col = jnp.arange(Np, dtype=idx.dtype)[:, None]
    return jnp.concatenate([idx, self_col], axis=1)


def kernel(x, pos, params):
    p128 = jnp.pad(pos, ((0, 0), (0, 125)))
    h = _mlp_bn(x, params['mlp_input'])
    nbrs = _graph_nbrs(pos)
    h = _tblock(params['t_in'], h, pos, p128[:, :16], nbrs)
    out_x, out_pos, out_p128, out_nbrs = [h], [pos], [p128], [nbrs]
    p = pos
    for i in range(4):
        n_out = int(math.ceil(p.shape[0] * 0.25))
        sel = _fps(p, n_out)
        p128 = _gather_rows(p128, sel)
        p_sub = p128[:, :3]
        idx = _knn(p_sub, p.T, 16, False)
        hm = _mlp_bn(h, params['td'][i])
        C = hm.shape[1]
        Cp = max(C, 128)
        g = _gather_rows(_pad_cols(hm, Cp), idx.T.reshape(-1))
        h = _rowmax(g.reshape(16, n_out, Cp))[:, :C]
        p = p_sub
        nbrs = _graph_nbrs(p)
        h = _tblock(params['tf_down'][i], h, p, p128[:, :16], nbrs)
        out_x.append(h)
        out_pos.append(p)
        out_p128.append(p128)
        out_nbrs.append(nbrs)
    h = _linrelu(h, params['mlp_summit'])
    h = _tblock(params['t_summit'], h, p, p128[:, :16], out_nbrs[-1])
    for i in range(4):
        x_skip = out_x[-i - 2]
        pos_skip = out_pos[-i - 2]
        pos_sub = out_pos[-i - 1]
        tu = params['tu'][3 - i]
        h_sub = _mlp_bn(h, tu['mlp_sub'])
        idx3 = _knn(pos_skip, pos_sub.T, 3, False)
        Np = pos_skip.shape[0]
        C = h_sub.shape[1]
        Gp = -(-(C + 16) // 128) * 128
        pack = _pad_cols(
            jnp.concatenate([h_sub, out_p128[-i - 1][:, :16]], axis=1), Gp)
        rows = _gather_rows(pack, idx3.T.reshape(-1))
        xg = rows[:, :C].reshape(3, Np, C)
        pg = rows[:, C:C + 3].reshape(3, Np, 3)
        base = _mlp_bn(x_skip, tu['mlp'])
        h = _interp(base, xg, pg, pos_skip)
        h = _tblock(params['tf_up'][3 - i], h, pos_skip,
                    out_p128[-i - 2][:, :16], out_nbrs[-i - 2])
    return _mlp2_plain(h, params['mlp_out'])


# speedup vs baseline: 1.3906x; 1.0828x over previous
import functools
import math

import jax
import jax.numpy as jnp
from jax import lax
from jax.experimental import pallas as pl
from jax.experimental.pallas import tpu as pltpu
from jax.experimental.pallas import tpu_sc as plsc

_INF = float('inf')


def _r7(v):
    u = lax.bitcast_convert_type(v, jnp.int32)
    u = (u + 32767 + ((u >> 16) & 1)) & jnp.int32(-65536)
    return lax.bitcast_convert_type(u, jnp.float32)


_NW = 32


def _sc_gather(table, idx):
    B = idx.shape[0]
    D = table.shape[1]
    bpw = B // _NW
    ch = min(128, bpw)
    while bpw % ch:
        ch -= 8
    n_ch = bpw // ch
    mesh = plsc.VectorSubcoreMesh(core_axis_name="c", subcore_axis_name="s")

    @functools.partial(
        pl.kernel, mesh=mesh,
        out_type=jax.ShapeDtypeStruct((B, D), jnp.float32),
        scratch_types=[
            pltpu.VMEM((ch,), jnp.int32),
            pltpu.VMEM((ch, D), jnp.float32),
            pltpu.SemaphoreType.DMA,
        ],
    )
    def k(table_hbm, idx_hbm, out_hbm, idx_v, rows_v, sem):
        wid = lax.axis_index("s") * 2 + lax.axis_index("c")
        base = wid * bpw

        def body(g, carry):
            off = base + g * ch
            pltpu.sync_copy(idx_hbm.at[pl.ds(off, ch)], idx_v)
            pltpu.async_copy(table_hbm.at[idx_v], rows_v, sem).wait()
            pltpu.sync_copy(rows_v, out_hbm.at[pl.ds(off, ch)])
            return carry

        lax.fori_loop(0, n_ch, body, jnp.int32(0))

    return k(table, idx)


def _gather_rows(table, idx):
    B0 = idx.shape[0]
    Bp = -(-B0 // 256) * 256
    if Bp != B0:
        idx = jnp.concatenate(
            [idx, jnp.zeros((Bp - B0,), idx.dtype)])
    out = _sc_gather(table, idx)
    return out[:B0] if Bp != B0 else out


def _knn_body(qref, rref, oref, *, k, R, TQ, exclude):
    t = pl.program_id(0)
    q = qref[...]
    rt = rref[...]
    qx, qy, qz = q[:, 0:1], q[:, 1:2], q[:, 2:3]
    rx, ry, rz = rt[0:1, :], rt[1:2, :], rt[2:3, :]
    qn = (qx * qx + qy * qy) + qz * qz
    rn = (rx * rx + ry * ry) + rz * rz
    qx7, qy7, qz7 = _r7(qx), _r7(qy), _r7(qz)
    rx7, ry7, rz7 = _r7(rx), _r7(ry), _r7(rz)
    m = (qx7 * rx7 + qy7 * ry7) + qz7 * rz7
    d = (qn + rn) - 2.0 * m
    cols = lax.broadcasted_iota(jnp.int32, (TQ, R), 1)
    if exclude:
        rows = lax.broadcasted_iota(jnp.int32, (TQ, R), 0) + t * TQ
        d = jnp.where(cols == rows, _INF, d)
    outs = []
    for _ in range(k):
        a = jnp.argmin(d, axis=1).astype(jnp.int32)[:, None]
        outs.append(a)
        d = jnp.where(cols == a, _INF, d)
    oref[...] = jnp.concatenate(outs, axis=1)


def _knn(query, refT, k, exclude):
    Q = query.shape[0]
    R = refT.shape[1]
    TQ = min(64, Q)
    body = functools.partial(_knn_body, k=k, R=R, TQ=TQ, exclude=exclude)
    return pl.pallas_call(
        body,
        grid=(Q // TQ,),
        in_specs=[pl.BlockSpec((TQ, 3), lambda i: (i, 0)),
                  pl.BlockSpec((3, R), lambda i: (0, 0))],
        out_specs=pl.BlockSpec((TQ, k), lambda i: (i, 0)),
        out_shape=jax.ShapeDtypeStruct((Q, k), jnp.int32),
    )(query, refT)


def _fps_body(pxr, pyr, pzr, oref, *, n_out, SR, SC, OR):
    px, py, pz = pxr[...], pyr[...], pzr[...]
    lin = (lax.broadcasted_iota(jnp.int32, (SR, SC), 0) * SC
           + lax.broadcasted_iota(jnp.int32, (SR, SC), 1))
    lane = lax.broadcasted_iota(jnp.int32, (1, SC), 1)
    oref[...] = jnp.zeros((OR, SC), jnp.int32)

    def body(i, carry):
        dists, last = carry
        msk = lin == last
        cx = jnp.sum(jnp.where(msk, px, 0.0))
        cy = jnp.sum(jnp.where(msk, py, 0.0))
        cz = jnp.sum(jnp.where(msk, pz, 0.0))
        dx, dy, dz = px - cx, py - cy, pz - cz
        d = (dx * dx + dy * dy) + dz * dz
        dists = jnp.minimum(dists, d)
        mx = jnp.max(dists)
        nxt = jnp.min(jnp.where(dists == mx, lin, jnp.int32(2147483647)))
        r = i // SC
        c = i % SC
        row = oref[pl.ds(r, 1), :]
        oref[pl.ds(r, 1), :] = jnp.where(lane == c, nxt, row)
        return dists, nxt

    dists0 = jnp.full((SR, SC), _INF, jnp.float32)
    lax.fori_loop(1, n_out, body, (dists0, jnp.int32(0)))


def _fps(p, n_out):
    Np = p.shape[0]
    SC = 128 if Np % 128 == 0 else Np
    SR = Np // SC
    OR = -(-n_out // SC)
    body = functools.partial(_fps_body, n_out=n_out, SR=SR, SC=SC, OR=OR)
    out = pl.pallas_call(
        body,
        out_shape=jax.ShapeDtypeStruct((OR, SC), jnp.int32),
    )(p[:, 0].reshape(SR, SC), p[:, 1].reshape(SR, SC),
      p[:, 2].reshape(SR, SC))
    return out.reshape(-1)[:n_out]


def _mlp_bn_body(xr, wr, br, gr, betr, oref):
    h = jnp.dot(xr[...], wr[...], preferred_element_type=jnp.float32) + br[...]
    mu = jnp.mean(h, axis=0, keepdims=True)
    xc = h - mu
    var = jnp.mean(xc * xc, axis=0, keepdims=True)
    h = xc / jnp.sqrt(var + 1e-5) * gr[...] + betr[...]
    oref[...] = jnp.maximum(h, 0.0)


def _mlp_bn(x, p):
    N = x.shape[0]
    dout = p['W'].shape[1]
    return pl.pallas_call(
        _mlp_bn_body,
        out_shape=jax.ShapeDtypeStruct((N, dout), jnp.float32),
    )(x, p['W'], p['b'].reshape(1, -1), p['gamma'].reshape(1, -1),
      p['beta'].reshape(1, -1))


def _linrelu_body(xr, wr, br, oref):
    h = jnp.dot(xr[...], wr[...], preferred_element_type=jnp.float32) + br[...]
    oref[...] = jnp.maximum(h, 0.0)


def _linrelu(x, p):
    N = x.shape[0]
    dout = p['W'].shape[1]
    return pl.pallas_call(
        _linrelu_body,
        out_shape=jax.ShapeDtypeStruct((N, dout), jnp.float32),
    )(x, p['W'], p['b'].reshape(1, -1))


def _mlp2_body(xr, w1r, b1r, w2r, b2r, oref):
    h = jnp.maximum(
        jnp.dot(xr[...], w1r[...], preferred_element_type=jnp.float32)
        + b1r[...], 0.0)
    oref[...] = (jnp.dot(h, w2r[...], preferred_element_type=jnp.float32)
                 + b2r[...])


def _mlp2_plain(x, p):
    N = x.shape[0]
    dout = p['l2']['W'].shape[1]
    return pl.pallas_call(
        _mlp2_body,
        out_shape=jax.ShapeDtypeStruct((N, dout), jnp.float32),
    )(x, p['l1']['W'], p['l1']['b'].reshape(1, -1),
      p['l2']['W'], p['l2']['b'].reshape(1, -1))


def _tpre_body(xr, wir, bir, wlr, wsr, wdr, vr, sr, dr):
    x1 = jnp.maximum(
        jnp.dot(xr[...], wir[...], preferred_element_type=jnp.float32)
        + bir[...], 0.0)
    vr[...] = jnp.dot(x1, wlr[...], preferred_element_type=jnp.float32)
    sr[...] = jnp.dot(x1, wsr[...], preferred_element_type=jnp.float32)
    dr[...] = jnp.dot(x1, wdr[...], preferred_element_type=jnp.float32)


def _tpre(x, p):
    N = x.shape[0]
    C = p['lin']['W'].shape[1]
    sh = jax.ShapeDtypeStruct((N, C), jnp.float32)
    return pl.pallas_call(
        _tpre_body,
        out_shape=[sh, sh, sh],
    )(x, p['lin_in']['W'], p['lin_in']['b'].reshape(1, -1),
      p['lin']['W'], p['lin_src']['W'], p['lin_dst']['W'])


def _edge_body(posr, pjr, xjr, ajr, adr, w1r, c1r, w2r, c2r,
               a1r, d1r, a2r, d2r, wor, cor, oref, *, K17):
    pos = posr[...]
    adst = adr[...]
    W1, b1 = w1r[...], c1r[...]
    W2, b2 = w2r[...], c2r[...]
    A1, e1 = a1r[...], d1r[...]
    A2, e2 = a2r[...], d2r[...]
    TP = pos.shape[0]
    C = adst.shape[1]
    E = K17 * TP
    pj = pjr[...].reshape(E, 3)
    xj = xjr[...].reshape(E, C)
    aj = ajr[...].reshape(E, C)
    posb = jnp.concatenate([pos] * K17, axis=0)
    adb = jnp.concatenate([adst] * K17, axis=0)
    pd = posb - pj
    h1 = jnp.maximum(
        jnp.dot(pd, W1, preferred_element_type=jnp.float32) + b1, 0.0)
    dl = jnp.maximum(
        jnp.dot(h1, W2, preferred_element_type=jnp.float32) + b2, 0.0)
    ai = (adb - aj) + dl
    h2 = jnp.maximum(
        jnp.dot(ai, A1, preferred_element_type=jnp.float32) + e1, 0.0)
    al = jnp.maximum(
        jnp.dot(h2, A2, preferred_element_type=jnp.float32) + e2, 0.0)
    m = al[0:TP]
    for j in range(1, K17):
        m = jnp.maximum(m, al[j * TP:(j + 1) * TP])
    es = [jnp.exp(al[j * TP:(j + 1) * TP] - m) for j in range(K17)]
    s = es[0]
    for j in range(1, K17):
        s = s + es[j]
    con = xj + dl
    o = (es[0] / s) * con[0:TP]
    for j in range(1, K17):
        o = o + (es[j] / s) * con[j * TP:(j + 1) * TP]
    oref[...] = jnp.maximum(
        jnp.dot(o, wor[...], preferred_element_type=jnp.float32) + cor[...],
        0.0)


def _edge(pos, pj, xj, aj, adst, p):
    Np, C = adst.shape
    K17 = pj.shape[0]
    TP = min(512, Np)
    body = functools.partial(_edge_body, K17=K17)
    return pl.pallas_call(
        body,
        grid=(Np // TP,),
        in_specs=[
            pl.BlockSpec((TP, 3), lambda i: (i, 0)),
            pl.BlockSpec((K17, TP, 3), lambda i: (0, i, 0)),
            pl.BlockSpec((K17, TP, C), lambda i: (0, i, 0)),
            pl.BlockSpec((K17, TP, C), lambda i: (0, i, 0)),
            pl.BlockSpec((TP, C), lambda i: (i, 0)),
            pl.BlockSpec((3, 64), lambda i: (0, 0)),
            pl.BlockSpec((1, 64), lambda i: (0, 0)),
            pl.BlockSpec((64, C), lambda i: (0, 0)),
            pl.BlockSpec((1, C), lambda i: (0, 0)),
            pl.BlockSpec((C, 64), lambda i: (0, 0)),
            pl.BlockSpec((1, 64), lambda i: (0, 0)),
            pl.BlockSpec((64, C), lambda i: (0, 0)),
            pl.BlockSpec((1, C), lambda i: (0, 0)),
            pl.BlockSpec((C, C), lambda i: (0, 0)),
            pl.BlockSpec((1, C), lambda i: (0, 0)),
        ],
        out_specs=pl.BlockSpec((TP, C), lambda i: (i, 0)),
        out_shape=jax.ShapeDtypeStruct((Np, C), jnp.float32),
    )(pos, pj, xj, aj, adst,
      p['pos_nn']['l1']['W'], p['pos_nn']['l1']['b'].reshape(1, -1),
      p['pos_nn']['l2']['W'], p['pos_nn']['l2']['b'].reshape(1, -1),
      p['attn_nn']['l1']['W'], p['attn_nn']['l1']['b'].reshape(1, -1),
      p['attn_nn']['l2']['W'], p['attn_nn']['l2']['b'].reshape(1, -1),
      p['lin_out']['W'], p['lin_out']['b'].reshape(1, -1))


def _pad_cols(a, W):
    return a if a.shape[1] == W else jnp.pad(a, ((0, 0), (0, W - a.shape[1])))


def _tblock(p, h, pos, p16, nbrs):
    v, asrc, adst = _tpre(h, p)
    Np, C = adst.shape
    K17 = nbrs.shape[1]
    Gp = -(-(2 * C + 16) // 128) * 128
    pack = _pad_cols(jnp.concatenate([v, asrc, p16], axis=1), Gp)
    rows = _gather_rows(pack, nbrs.T.reshape(-1))
    xj = rows[:, :C].reshape(K17, Np, C)
    aj = rows[:, C:2 * C].reshape(K17, Np, C)
    pj = rows[:, 2 * C:2 * C + 3].reshape(K17, Np, 3)
    return _edge(pos, pj, xj, aj, adst, p)


def _rowmax_body(gr, oref, *, KK):
    m = gr[0]
    for j in range(1, KK):
        m = jnp.maximum(m, gr[j])
    oref[...] = m


def _rowmax(g):
    KK, Ns, C = g.shape
    TP = min(512, Ns)
    body = functools.partial(_rowmax_body, KK=KK)
    return pl.pallas_call(
        body,
        grid=(Ns // TP,),
        in_specs=[pl.BlockSpec((KK, TP, C), lambda i: (0, i, 0))],
        out_specs=pl.BlockSpec((TP, C), lambda i: (i, 0)),
        out_shape=jax.ShapeDtypeStruct((Ns, C), jnp.float32),
    )(g)


def _interp_body(baser, xgr, pgr, posr, oref):
    pos = posr[...]
    num = None
    den = None
    for j in range(3):
        pd = pos - pgr[j]
        dx, dy, dz = pd[:, 0:1], pd[:, 1:2], pd[:, 2:3]
        d2 = (dx * dx + dy * dy) + dz * dz
        w = 1.0 / jnp.maximum(d2, 1e-16)
        contrib = xgr[j] * w
        num = contrib if num is None else num + contrib
        den = w if den is None else den + w
    oref[...] = baser[...] + num / den


def _interp(base, xg, pg, pos):
    Np, C = base.shape
    TP = min(512, Np)
    return pl.pallas_call(
        _interp_body,
        grid=(Np // TP,),
        in_specs=[
            pl.BlockSpec((TP, C), lambda i: (i, 0)),
            pl.BlockSpec((3, TP, C), lambda i: (0, i, 0)),
            pl.BlockSpec((3, TP, 3), lambda i: (0, i, 0)),
            pl.BlockSpec((TP, 3), lambda i: (i, 0)),
        ],
        out_specs=pl.BlockSpec((TP, C), lambda i: (i, 0)),
        out_shape=jax.ShapeDtypeStruct((Np, C), jnp.float32),
    )(base, xg, pg, pos)


def _graph_nbrs(pos):
    Np = pos.shape[0]
    idx = _knn(pos, pos.T, 16, True)
    self_col = jnp.arange(Np, dtype=idx.dtype)[:, None]
    return jnp.concatenate([idx, self_col], axis=1)


def kernel(x, pos, params):
    p128 = jnp.pad(pos, ((0, 0), (0, 125)))
    h = _mlp_bn(x, params['mlp_input'])
    nbrs = _graph_nbrs(pos)
    h = _tblock(params['t_in'], h, pos, p128[:, :16], nbrs)
    out_x, out_pos, out_p128, out_nbrs = [h], [pos], [p128], [nbrs]
    p = pos
    for i in range(4):
        n_out = int(math.ceil(p.shape[0] * 0.25))
        sel = _fps(p, n_out)
        p128 = _gather_rows(p128, sel)
        p_sub = p128[:, :3]
        idx = _knn(p_sub, p.T, 16, False)
        hm = _mlp_bn(h, params['td'][i])
        C = hm.shape[1]
        Cp = max(C, 128)
        g = _gather_rows(_pad_cols(hm, Cp), idx.T.reshape(-1))
        h = _rowmax(g.reshape(16, n_out, Cp))[:, :C]
        p = p_sub
        nbrs = _graph_nbrs(p)
        h = _tblock(params['tf_down'][i], h, p, p128[:, :16], nbrs)
        out_x.append(h)
        out_pos.append(p)
        out_p128.append(p128)
        out_nbrs.append(nbrs)
    h = _linrelu(h, params['mlp_summit'])
    h = _tblock(params['t_summit'], h, p, p128[:, :16], out_nbrs[-1])
    for i in range(4):
        x_skip = out_x[-i - 2]
        pos_skip = out_pos[-i - 2]
        pos_sub = out_pos[-i - 1]
        tu = params['tu'][3 - i]
        h_sub = _mlp_bn(h, tu['mlp_sub'])
        idx3 = _knn(pos_skip, pos_sub.T, 3, False)
        Np = pos_skip.shape[0]
        C = h_sub.shape[1]
        Gp = -(-(C + 16) // 128) * 128
        pack = _pad_cols(
            jnp.concatenate([h_sub, out_p128[-i - 1][:, :16]], axis=1), Gp)
        rows = _gather_rows(pack, idx3.T.reshape(-1))
        xg = rows[:, :C].reshape(3, Np, C)
        pg = rows[:, C:C + 3].reshape(3, Np, 3)
        base = _mlp_bn(x_skip, tu['mlp'])
        h = _interp(base, xg, pg, pos_skip)
        h = _tblock(params['tf_up'][3 - i], h, pos_skip,
                    out_p128[-i - 2][:, :16], out_nbrs[-i - 2])
    return _mlp2_plain(h, params['mlp_out'])
